# baseline (speedup 1.0000x reference)
import jax
import jax.numpy as jnp
from jax import lax
from jax.experimental import pallas as pl
from jax.experimental.pallas import tpu as pltpu
from jax.experimental.pallas import tpu_sc as plsc

N = 100000
C = 256
CO = C // 2
SC_ROWS = 48000
TC_ROWS = N - SC_ROWS
BR = 2000
NC, NS = 2, 16
NW = NC * NS
TILE_ROWS = SC_ROWS // NW
R = 125
CHUNKS = TILE_ROWS // R
IN_CHUNK = R * C
OUT_CHUNK = R * CO
VECS = OUT_CHUNK // 16
NBI = 3
NBO = 2


def _sc_body(feat_ref, out_ref, in0, in1, in2, ot0, ot1,
             si0, si1, si2, so0, so1):
    wid = lax.axis_index("s") * NC + lax.axis_index("c")
    in_base = wid * (TILE_ROWS * C)
    out_base = wid * (TILE_ROWS * CO)
    iota2 = lax.iota(jnp.int32, 16) * 2

    in_b = (in0, in1, in2)
    out_b = (ot0, ot1)
    si = (si0, si1, si2)
    so = (so0, so1)

    def start_in(c):
        b = c % NBI
        return pltpu.async_copy(
            feat_ref.at[pl.ds(in_base + c * IN_CHUNK, IN_CHUNK)], in_b[b], si[b]
        )

    in_h = [None] * NBI
    out_h = [None] * NBO
    for c in range(min(NBI - 1, CHUNKS)):
        in_h[c % NBI] = start_in(c)
    for c in range(CHUNKS):
        bi = c % NBI
        bo = c % NBO
        if c + NBI - 1 < CHUNKS:
            in_h[(c + NBI - 1) % NBI] = start_in(c + NBI - 1)
        in_h[bi].wait()
        if out_h[bo] is not None:
            out_h[bo].wait()
        src = in_b[bi]
        dst = out_b[bo]

        @plsc.parallel_loop(0, VECS, 1, unroll=8, carry=iota2)
        def _chunk(k, idx):
            dst[pl.ds(k * 16, 16)] = plsc.load_gather(src, [idx])
            return idx + 32

        out_h[bo] = pltpu.async_copy(
            dst, out_ref.at[pl.ds(out_base + c * OUT_CHUNK, OUT_CHUNK)], so[bo]
        )
    for bo in range(NBO):
        if out_h[bo] is not None:
            out_h[bo].wait()


@jax.jit
def kernel(features):
    flat = features[TC_ROWS:].reshape(-1)
    run = pl.kernel(
        _sc_body,
        out_type=jax.ShapeDtypeStruct((SC_ROWS * CO,), jnp.float32),
        mesh=plsc.VectorSubcoreMesh(core_axis_name="c", subcore_axis_name="s"),
        compiler_params=pltpu.CompilerParams(needs_layout_passes=False),
        scratch_types=[
            pltpu.VMEM((IN_CHUNK,), jnp.float32),
            pltpu.VMEM((IN_CHUNK,), jnp.float32),
            pltpu.VMEM((IN_CHUNK,), jnp.float32),
            pltpu.VMEM((OUT_CHUNK,), jnp.float32),
            pltpu.VMEM((OUT_CHUNK,), jnp.float32),
            pltpu.SemaphoreType.DMA,
            pltpu.SemaphoreType.DMA,
            pltpu.SemaphoreType.DMA,
            pltpu.SemaphoreType.DMA,
            pltpu.SemaphoreType.DMA,
        ],
    )
    bottom = run(flat).reshape(SC_ROWS, CO)

    import numpy as np
    sel = jnp.asarray(np.eye(C, dtype=np.float32)[:, ::2])

    def _tc_body(x_ref, s_ref, o_ref):
        o_ref[...] = jax.lax.dot(
            x_ref[...], s_ref[...], preferred_element_type=jnp.float32
        )

    top = pl.pallas_call(
        _tc_body,
        grid=(TC_ROWS // BR,),
        in_specs=[
            pl.BlockSpec((BR, C), lambda i: (i, 0)),
            pl.BlockSpec((C, CO), lambda i: (0, 0)),
        ],
        out_specs=pl.BlockSpec((BR, CO), lambda i: (i, 0)),
        out_shape=jax.ShapeDtypeStruct((TC_ROWS, CO), jnp.float32),
    )(features[:TC_ROWS], sel)
    return jnp.concatenate([top, bottom], axis=0)


# speedup vs baseline: 1.3953x; 1.3953x over previous
import jax
import jax.numpy as jnp
from jax import lax
from jax.experimental import pallas as pl
from jax.experimental.pallas import tpu as pltpu
from jax.experimental.pallas import tpu_sc as plsc

N = 100000
C = 256
CO = C // 2
NC, NS = 2, 16
NW = NC * NS
TILE_ROWS = N // NW
R = 125
CHUNKS = TILE_ROWS // R
IN_CHUNK = R * C
OUT_CHUNK = R * CO
VECS = OUT_CHUNK // 16
NBI = 3
NBO = 2


def _sc_body(feat_ref, out_ref, in0, in1, in2, ot0, ot1,
             si0, si1, si2, so0, so1):
    wid = lax.axis_index("s") * NC + lax.axis_index("c")
    in_base = wid * (TILE_ROWS * C)
    out_base = wid * (TILE_ROWS * CO)
    iota2 = lax.iota(jnp.int32, 16) * 2

    in_b = (in0, in1, in2)
    out_b = (ot0, ot1)
    si = (si0, si1, si2)
    so = (so0, so1)

    def start_in(c):
        b = c % NBI
        return pltpu.async_copy(
            feat_ref.at[pl.ds(in_base + c * IN_CHUNK, IN_CHUNK)], in_b[b], si[b]
        )

    in_h = [None] * NBI
    out_h = [None] * NBO
    for c in range(min(NBI - 1, CHUNKS)):
        in_h[c % NBI] = start_in(c)
    for c in range(CHUNKS):
        bi = c % NBI
        bo = c % NBO
        if c + NBI - 1 < CHUNKS:
            in_h[(c + NBI - 1) % NBI] = start_in(c + NBI - 1)
        in_h[bi].wait()
        if out_h[bo] is not None:
            out_h[bo].wait()
        src = in_b[bi]
        dst = out_b[bo]

        @plsc.parallel_loop(0, VECS, 1, unroll=8, carry=iota2)
        def _chunk(k, idx):
            dst[pl.ds(k * 16, 16)] = plsc.load_gather(src, [idx])
            return idx + 32

        out_h[bo] = pltpu.async_copy(
            dst, out_ref.at[pl.ds(out_base + c * OUT_CHUNK, OUT_CHUNK)], so[bo]
        )
    for bo in range(NBO):
        if out_h[bo] is not None:
            out_h[bo].wait()


@jax.jit
def kernel(features):
    flat = features.reshape(-1)
    run = pl.kernel(
        _sc_body,
        out_type=jax.ShapeDtypeStruct((N * CO,), jnp.float32),
        mesh=plsc.VectorSubcoreMesh(core_axis_name="c", subcore_axis_name="s"),
        compiler_params=pltpu.CompilerParams(needs_layout_passes=False),
        scratch_types=[
            pltpu.VMEM((IN_CHUNK,), jnp.float32),
            pltpu.VMEM((IN_CHUNK,), jnp.float32),
            pltpu.VMEM((IN_CHUNK,), jnp.float32),
            pltpu.VMEM((OUT_CHUNK,), jnp.float32),
            pltpu.VMEM((OUT_CHUNK,), jnp.float32),
            pltpu.SemaphoreType.DMA,
            pltpu.SemaphoreType.DMA,
            pltpu.SemaphoreType.DMA,
            pltpu.SemaphoreType.DMA,
            pltpu.SemaphoreType.DMA,
        ],
    )
    return run(flat).reshape(N, CO)
